# Initial kernel scaffold; baseline (speedup 1.0000x reference)
import jax
import jax.numpy as jnp
from jax.experimental import pallas as pl


def kernel(feats, coors, W_e1, b_e1, W_e2, b_e2, W_n1, b_n1, W_n2, b_n2, W_c1, b_c1, W_c2, b_c2, W_x1, b_x1, W_x2, b_x2):
    raise NotImplementedError("write your pallas kernel here")


# speedup vs baseline: 6.8031x; 6.8031x over previous
import functools

import jax
import jax.numpy as jnp
from jax.experimental import pallas as pl
from jax.experimental.pallas import tpu as pltpu

K = 32
R = 64
IDX_MASK = 1023


def _silu(x):
    return x * jax.nn.sigmoid(x)


def _roll3(x, shift):
    if shift == -1:
        return jnp.concatenate([x[:, 1:3], x[:, 0:1]], axis=1)
    return jnp.concatenate([x[:, 2:3], x[:, 0:2]], axis=1)


def _egnn_block(feats_ref, coors_ref, coors_t_ref,
                W1a_ref, W1b_ref, w1c_ref, be1_ref, We2_ref, be2_ref,
                Wn1a_ref, Wn1b_ref, bn1_ref, Wn2_ref, bn2_ref,
                Wcx1_ref, bcx1_ref, Wcx2_ref, bcx2_ref,
                node_out_ref, coors_out_ref):
    n = feats_ref.shape[1]
    i0 = pl.program_id(1) * R
    E = R * K

    feats_full = feats_ref[0]
    coors_full = coors_ref[0]
    feats_blk = feats_ref[0, pl.ds(i0, R), :]
    coors_blk = coors_ref[0, pl.ds(i0, R), :]

    cxj = coors_t_ref[0, 0:1, :]
    cyj = coors_t_ref[0, 1:2, :]
    czj = coors_t_ref[0, 2:3, :]
    dx = coors_blk[:, 0:1] - cxj
    dy = coors_blk[:, 1:2] - cyj
    dz = coors_blk[:, 2:3] - czj
    d2 = dx * dx + dy * dy + dz * dz

    bits = jax.lax.bitcast_convert_type(d2, jnp.int32)
    jcol = jax.lax.broadcasted_iota(jnp.int32, (R, n), 1)
    arr = (bits & jnp.int32(~IDX_MASK)) | jcol
    maxval = jnp.int32(0x7FFFFFFF)
    cols = []
    for _ in range(K):
        m = jnp.min(arr, axis=1, keepdims=True)
        cols.append(m)
        arr = jnp.where(arr == m, maxval, arr)
    packed_flat = jnp.concatenate(cols, axis=0)
    idx_flat = packed_flat & IDX_MASK
    dist_flat = jax.lax.bitcast_convert_type(
        packed_flat & jnp.int32(~IDX_MASK), jnp.float32)

    mean_c = jnp.mean(coors_full, axis=0, keepdims=True)
    cnm = coors_full - mean_c
    aug = jnp.concatenate([feats_full, coors_full, cnm],
                          axis=1).astype(jnp.bfloat16)

    lane = jax.lax.broadcasted_iota(jnp.int32, (E, n), 1)
    sub_i = jax.lax.broadcasted_iota(jnp.int32, (E, n), 0) % R + i0
    oh_j = (lane == idx_flat).astype(jnp.bfloat16)
    oh_i = (lane == sub_i).astype(jnp.bfloat16)
    Gj = jnp.dot(oh_j, aug, preferred_element_type=jnp.float32)
    Gi = jnp.dot(oh_i, aug, preferred_element_type=jnp.float32)

    fi = Gi[:, 0:64].astype(jnp.bfloat16)
    fj = Gj[:, 0:64].astype(jnp.bfloat16)
    h = (jnp.dot(fi, W1a_ref[...], preferred_element_type=jnp.float32)
         + jnp.dot(fj, W1b_ref[...], preferred_element_type=jnp.float32)
         + dist_flat * w1c_ref[...]
         + be1_ref[...])
    h = _silu(h)
    m_ij = _silu(jnp.dot(h.astype(jnp.bfloat16), We2_ref[...],
                         preferred_element_type=jnp.float32)
                 + be2_ref[...])

    t12 = _silu(jnp.dot(m_ij.astype(jnp.bfloat16), Wcx1_ref[...],
                        preferred_element_type=jnp.float32)
                + bcx1_ref[...])
    cw2 = (jnp.dot(t12.astype(jnp.bfloat16), Wcx2_ref[...],
                   preferred_element_type=jnp.float32)
           + bcx2_ref[...])
    cw = cw2[:, 0:1]
    cwx = cw2[:, 1:2]

    rel = Gi[:, 64:67] - Gj[:, 64:67]
    ai = Gi[:, 67:70]
    bj = Gj[:, 67:70]
    cross = _roll3(ai, -1) * _roll3(bj, 1) - _roll3(ai, 1) * _roll3(bj, -1)
    contrib = cw * rel + cwx * cross

    pool_lane = jax.lax.broadcasted_iota(jnp.int32, (R, E), 1) % R
    pool_sub = jax.lax.broadcasted_iota(jnp.int32, (R, E), 0)
    pool = (pool_lane == pool_sub).astype(jnp.bfloat16)
    pooled = jnp.dot(pool,
                     jnp.concatenate([contrib, m_ij], axis=1).astype(jnp.bfloat16),
                     preferred_element_type=jnp.float32)
    csum = pooled[:, 0:3]
    m_i = pooled[:, 3:19]

    coors_out_ref[0] = csum + coors_blk

    nh = _silu(jnp.dot(feats_blk.astype(jnp.bfloat16), Wn1a_ref[...],
                       preferred_element_type=jnp.float32)
               + jnp.dot(m_i.astype(jnp.bfloat16), Wn1b_ref[...],
                         preferred_element_type=jnp.float32)
               + bn1_ref[...])
    node = (jnp.dot(nh.astype(jnp.bfloat16), Wn2_ref[...],
                    preferred_element_type=jnp.float32)
            + bn2_ref[...] + feats_blk)
    node_out_ref[0] = node


@jax.jit
def kernel(feats, coors, W_e1, b_e1, W_e2, b_e2, W_n1, b_n1, W_n2, b_n2,
           W_c1, b_c1, W_c2, b_c2, W_x1, b_x1, W_x2, b_x2):
    b, n, d = feats.shape
    m_dim = W_e2.shape[1]

    coors_t = jnp.transpose(coors, (0, 2, 1))

    bf = jnp.bfloat16
    W1a = W_e1[:d].astype(bf)
    W1b = W_e1[d:2 * d].astype(bf)
    w1c = W_e1[2 * d:2 * d + 1]
    be1 = b_e1[None, :]
    We2 = W_e2.astype(bf)
    be2 = b_e2[None, :]
    Wn1a = W_n1[:d].astype(bf)
    Wn1b = W_n1[d:d + m_dim].astype(bf)
    bn1 = b_n1[None, :]
    Wn2 = W_n2.astype(bf)
    bn2 = b_n2[None, :]
    Wcx1 = jnp.concatenate([W_c1, W_x1], axis=1).astype(bf)
    bcx1 = jnp.concatenate([b_c1, b_x1])[None, :]
    zeros = jnp.zeros_like(W_c2)
    Wcx2 = jnp.concatenate(
        [jnp.concatenate([W_c2, zeros], axis=1),
         jnp.concatenate([zeros, W_x2], axis=1)], axis=0).astype(bf)
    bcx2 = jnp.concatenate([b_c2, b_x2])[None, :]

    full = lambda shp: pl.BlockSpec(shp, lambda bi, ii: (0,) * len(shp))

    grid = (b, n // R)
    node_out, coors_out = pl.pallas_call(
        _egnn_block,
        grid=grid,
        in_specs=[
            pl.BlockSpec((1, n, d), lambda bi, ii: (bi, 0, 0)),
            pl.BlockSpec((1, n, 3), lambda bi, ii: (bi, 0, 0)),
            pl.BlockSpec((1, 3, n), lambda bi, ii: (bi, 0, 0)),
            full(W1a.shape), full(W1b.shape), full(w1c.shape), full(be1.shape),
            full(We2.shape), full(be2.shape),
            full(Wn1a.shape), full(Wn1b.shape), full(bn1.shape),
            full(Wn2.shape), full(bn2.shape),
            full(Wcx1.shape), full(bcx1.shape), full(Wcx2.shape),
            full(bcx2.shape),
        ],
        out_specs=[
            pl.BlockSpec((1, R, d), lambda bi, ii: (bi, ii, 0)),
            pl.BlockSpec((1, R, 3), lambda bi, ii: (bi, ii, 0)),
        ],
        out_shape=[
            jax.ShapeDtypeStruct((b, n, d), jnp.float32),
            jax.ShapeDtypeStruct((b, n, 3), jnp.float32),
        ],
    )(feats, coors, coors_t, W1a, W1b, w1c, be1, We2, be2,
      Wn1a, Wn1b, bn1, Wn2, bn2, Wcx1, bcx1, Wcx2, bcx2)
    return node_out, coors_out


# speedup vs baseline: 6.9608x; 1.0232x over previous
import functools

import jax
import jax.numpy as jnp
from jax.experimental import pallas as pl
from jax.experimental.pallas import tpu as pltpu

K = 32
R = 64
IDX_MASK = 1023


def _silu(x):
    return x * jax.nn.sigmoid(x)


def _roll3(x, shift):
    if shift == -1:
        return jnp.concatenate([x[:, 1:3], x[:, 0:1]], axis=1)
    return jnp.concatenate([x[:, 2:3], x[:, 0:2]], axis=1)


def _tile_k(x):
    return jnp.concatenate([x] * K, axis=0)


def _egnn_block(feats_ref, coors_ref, coors_t_ref, pool_ref,
                W1a_ref, W1b_ref, w1c_ref, be1_ref, We2_ref, be2_ref,
                Wn1a_ref, Wn1b_ref, bn1_ref, Wn2_ref, bn2_ref,
                Wcx1_ref, bcx1_ref, Wcx2_ref, bcx2_ref,
                node_out_ref, coors_out_ref):
    n = feats_ref.shape[1]
    i0 = pl.program_id(1) * R
    E = R * K

    feats_full = feats_ref[0]
    coors_full = coors_ref[0]
    feats_blk = feats_ref[0, pl.ds(i0, R), :]
    coors_blk = coors_ref[0, pl.ds(i0, R), :]

    cxj = coors_t_ref[0, 0:1, :]
    cyj = coors_t_ref[0, 1:2, :]
    czj = coors_t_ref[0, 2:3, :]
    dx = coors_blk[:, 0:1] - cxj
    dy = coors_blk[:, 1:2] - cyj
    dz = coors_blk[:, 2:3] - czj
    d2 = dx * dx + dy * dy + dz * dz

    bits = jax.lax.bitcast_convert_type(d2, jnp.int32)
    jcol = jax.lax.broadcasted_iota(jnp.int32, (R, n), 1)
    arr = (bits & jnp.int32(~IDX_MASK)) | jcol
    maxval = jnp.int32(0x7FFFFFFF)
    cols = []
    for _ in range(K):
        m = jnp.min(arr, axis=1, keepdims=True)
        cols.append(m)
        arr = jnp.where(arr == m, maxval, arr)
    packed_flat = jnp.concatenate(cols, axis=0)
    idx_flat = packed_flat & IDX_MASK
    dist_flat = jax.lax.bitcast_convert_type(
        packed_flat & jnp.int32(~IDX_MASK), jnp.float32)

    mean_c = jnp.mean(coors_full, axis=0, keepdims=True)
    cnm = coors_full - mean_c
    aug = jnp.concatenate([feats_full, coors_full, cnm],
                          axis=1).astype(jnp.bfloat16)

    lane = jax.lax.broadcasted_iota(jnp.int32, (E, n), 1)
    oh_j = (lane == idx_flat).astype(jnp.bfloat16)
    Gj = jnp.dot(oh_j, aug, preferred_element_type=jnp.float32)

    P_i = jnp.dot(feats_blk.astype(jnp.bfloat16), W1a_ref[...],
                  preferred_element_type=jnp.float32)
    fj = Gj[:, 0:64].astype(jnp.bfloat16)
    h = (_tile_k(P_i)
         + jnp.dot(fj, W1b_ref[...], preferred_element_type=jnp.float32)
         + dist_flat * w1c_ref[...]
         + be1_ref[...])
    h = _silu(h)
    m_ij = _silu(jnp.dot(h.astype(jnp.bfloat16), We2_ref[...],
                         preferred_element_type=jnp.float32)
                 + be2_ref[...])

    t12 = _silu(jnp.dot(m_ij.astype(jnp.bfloat16), Wcx1_ref[...],
                        preferred_element_type=jnp.float32)
                + bcx1_ref[...])
    cw2 = (jnp.dot(t12.astype(jnp.bfloat16), Wcx2_ref[...],
                   preferred_element_type=jnp.float32)
           + bcx2_ref[...])
    cw = cw2[:, 0:1]
    cwx = cw2[:, 1:2]

    rel = _tile_k(coors_blk) - Gj[:, 64:67]
    ai = _tile_k(coors_blk - mean_c)
    bj = Gj[:, 67:70]
    cross = _roll3(ai, -1) * _roll3(bj, 1) - _roll3(ai, 1) * _roll3(bj, -1)
    contrib = cw * rel + cwx * cross

    pooled = jnp.dot(pool_ref[...],
                     jnp.concatenate([contrib, m_ij], axis=1).astype(jnp.bfloat16),
                     preferred_element_type=jnp.float32)
    csum = pooled[:, 0:3]
    m_i = pooled[:, 3:19]

    coors_out_ref[0] = csum + coors_blk

    nh = _silu(jnp.dot(feats_blk.astype(jnp.bfloat16), Wn1a_ref[...],
                       preferred_element_type=jnp.float32)
               + jnp.dot(m_i.astype(jnp.bfloat16), Wn1b_ref[...],
                         preferred_element_type=jnp.float32)
               + bn1_ref[...])
    node = (jnp.dot(nh.astype(jnp.bfloat16), Wn2_ref[...],
                    preferred_element_type=jnp.float32)
            + bn2_ref[...] + feats_blk)
    node_out_ref[0] = node


@jax.jit
def kernel(feats, coors, W_e1, b_e1, W_e2, b_e2, W_n1, b_n1, W_n2, b_n2,
           W_c1, b_c1, W_c2, b_c2, W_x1, b_x1, W_x2, b_x2):
    b, n, d = feats.shape
    m_dim = W_e2.shape[1]

    coors_t = jnp.transpose(coors, (0, 2, 1))

    E = R * K
    pool = (jnp.arange(E, dtype=jnp.int32)[None, :] % R
            == jnp.arange(R, dtype=jnp.int32)[:, None]).astype(jnp.bfloat16)

    bf = jnp.bfloat16
    W1a = W_e1[:d].astype(bf)
    W1b = W_e1[d:2 * d].astype(bf)
    w1c = W_e1[2 * d:2 * d + 1]
    be1 = b_e1[None, :]
    We2 = W_e2.astype(bf)
    be2 = b_e2[None, :]
    Wn1a = W_n1[:d].astype(bf)
    Wn1b = W_n1[d:d + m_dim].astype(bf)
    bn1 = b_n1[None, :]
    Wn2 = W_n2.astype(bf)
    bn2 = b_n2[None, :]
    Wcx1 = jnp.concatenate([W_c1, W_x1], axis=1).astype(bf)
    bcx1 = jnp.concatenate([b_c1, b_x1])[None, :]
    zeros = jnp.zeros_like(W_c2)
    Wcx2 = jnp.concatenate(
        [jnp.concatenate([W_c2, zeros], axis=1),
         jnp.concatenate([zeros, W_x2], axis=1)], axis=0).astype(bf)
    bcx2 = jnp.concatenate([b_c2, b_x2])[None, :]

    full = lambda shp: pl.BlockSpec(shp, lambda bi, ii: (0,) * len(shp))

    grid = (b, n // R)
    node_out, coors_out = pl.pallas_call(
        _egnn_block,
        grid=grid,
        in_specs=[
            pl.BlockSpec((1, n, d), lambda bi, ii: (bi, 0, 0)),
            pl.BlockSpec((1, n, 3), lambda bi, ii: (bi, 0, 0)),
            pl.BlockSpec((1, 3, n), lambda bi, ii: (bi, 0, 0)),
            full(pool.shape),
            full(W1a.shape), full(W1b.shape), full(w1c.shape), full(be1.shape),
            full(We2.shape), full(be2.shape),
            full(Wn1a.shape), full(Wn1b.shape), full(bn1.shape),
            full(Wn2.shape), full(bn2.shape),
            full(Wcx1.shape), full(bcx1.shape), full(Wcx2.shape),
            full(bcx2.shape),
        ],
        out_specs=[
            pl.BlockSpec((1, R, d), lambda bi, ii: (bi, ii, 0)),
            pl.BlockSpec((1, R, 3), lambda bi, ii: (bi, ii, 0)),
        ],
        out_shape=[
            jax.ShapeDtypeStruct((b, n, d), jnp.float32),
            jax.ShapeDtypeStruct((b, n, 3), jnp.float32),
        ],
    )(feats, coors, coors_t, pool, W1a, W1b, w1c, be1, We2, be2,
      Wn1a, Wn1b, bn1, Wn2, bn2, Wcx1, bcx1, Wcx2, bcx2)
    return node_out, coors_out


# speedup vs baseline: 9.4500x; 1.3576x over previous
import functools

import jax
import jax.numpy as jnp
from jax.experimental import pallas as pl
from jax.experimental.pallas import tpu as pltpu

K = 32
R = 128
IDX_MASK = 1023


def _silu(x):
    return x * jax.nn.sigmoid(x)


def _roll3(x, shift):
    if shift == -1:
        return jnp.concatenate([x[:, 1:3], x[:, 0:1]], axis=1)
    return jnp.concatenate([x[:, 2:3], x[:, 0:2]], axis=1)


def _tile_k(x):
    return jnp.concatenate([x] * K, axis=0)


def _egnn_block(feats_ref, coors_ref, coors_t_ref, pool_ref,
                W1a_ref, W1b_ref, w1c_ref, be1_ref, We2_ref, be2_ref,
                Wn1a_ref, Wn1b_ref, bn1_ref, Wn2_ref, bn2_ref,
                Wcx1_ref, bcx1_ref, Wcx2_ref, bcx2_ref,
                node_out_ref, coors_out_ref):
    n = feats_ref.shape[1]
    i0 = pl.program_id(1) * R
    E = R * K

    feats_full = feats_ref[0]
    coors_full = coors_ref[0]
    feats_blk = feats_ref[0, pl.ds(i0, R), :]
    coors_blk = coors_ref[0, pl.ds(i0, R), :]

    cxj = coors_t_ref[0, 0:1, :]
    cyj = coors_t_ref[0, 1:2, :]
    czj = coors_t_ref[0, 2:3, :]
    dx = coors_blk[:, 0:1] - cxj
    dy = coors_blk[:, 1:2] - cyj
    dz = coors_blk[:, 2:3] - czj
    d2 = dx * dx + dy * dy + dz * dz

    bits = jax.lax.bitcast_convert_type(d2, jnp.int32)
    jcol = jax.lax.broadcasted_iota(jnp.int32, (R, n), 1)
    arr = (bits & jnp.int32(~IDX_MASK)) | jcol
    maxval = jnp.int32(0x7FFFFFFF)
    cols = []
    for _ in range(K):
        m = jnp.min(arr, axis=1, keepdims=True)
        cols.append(m)
        arr = jnp.where(arr == m, maxval, arr)
    packed_flat = jnp.concatenate(cols, axis=0)
    idx_flat = packed_flat & IDX_MASK
    dist_flat = jax.lax.bitcast_convert_type(
        packed_flat & jnp.int32(~IDX_MASK), jnp.float32)

    mean_c = jnp.mean(coors_full, axis=0, keepdims=True)
    cnm = coors_full - mean_c
    aug = jnp.concatenate([feats_full, coors_full, cnm],
                          axis=1).astype(jnp.bfloat16)

    lane = jax.lax.broadcasted_iota(jnp.int32, (E, n), 1)
    oh_j = (lane == idx_flat).astype(jnp.bfloat16)
    Gj = jnp.dot(oh_j, aug, preferred_element_type=jnp.float32)

    P_i = jnp.dot(feats_blk.astype(jnp.bfloat16), W1a_ref[...],
                  preferred_element_type=jnp.float32)
    fj = Gj[:, 0:64].astype(jnp.bfloat16)
    h = (_tile_k(P_i)
         + jnp.dot(fj, W1b_ref[...], preferred_element_type=jnp.float32)
         + dist_flat * w1c_ref[...]
         + be1_ref[...])
    h = _silu(h.astype(jnp.bfloat16))
    m_ij = _silu((jnp.dot(h, We2_ref[...],
                          preferred_element_type=jnp.float32)
                  + be2_ref[...]).astype(jnp.bfloat16))

    t12 = _silu((jnp.dot(m_ij, Wcx1_ref[...],
                         preferred_element_type=jnp.float32)
                 + bcx1_ref[...]).astype(jnp.bfloat16))
    cw2 = (jnp.dot(t12, Wcx2_ref[...],
                   preferred_element_type=jnp.float32)
           + bcx2_ref[...])
    cw = cw2[:, 0:1]
    cwx = cw2[:, 1:2]

    rel = _tile_k(coors_blk) - Gj[:, 64:67]
    ai = _tile_k(coors_blk - mean_c)
    bj = Gj[:, 67:70]
    cross = _roll3(ai, -1) * _roll3(bj, 1) - _roll3(ai, 1) * _roll3(bj, -1)
    contrib = cw * rel + cwx * cross

    pooled = jnp.dot(pool_ref[...],
                     jnp.concatenate([contrib.astype(jnp.bfloat16), m_ij],
                                     axis=1),
                     preferred_element_type=jnp.float32)
    csum = pooled[:, 0:3]
    m_i = pooled[:, 3:19]

    coors_out_ref[0] = csum + coors_blk

    nh = _silu((jnp.dot(feats_blk.astype(jnp.bfloat16), Wn1a_ref[...],
                        preferred_element_type=jnp.float32)
                + jnp.dot(m_i.astype(jnp.bfloat16), Wn1b_ref[...],
                          preferred_element_type=jnp.float32)
                + bn1_ref[...]).astype(jnp.bfloat16))
    node = (jnp.dot(nh, Wn2_ref[...],
                    preferred_element_type=jnp.float32)
            + bn2_ref[...] + feats_blk)
    node_out_ref[0] = node


@jax.jit
def kernel(feats, coors, W_e1, b_e1, W_e2, b_e2, W_n1, b_n1, W_n2, b_n2,
           W_c1, b_c1, W_c2, b_c2, W_x1, b_x1, W_x2, b_x2):
    b, n, d = feats.shape
    m_dim = W_e2.shape[1]

    coors_t = jnp.transpose(coors, (0, 2, 1))

    E = R * K
    pool = (jnp.arange(E, dtype=jnp.int32)[None, :] % R
            == jnp.arange(R, dtype=jnp.int32)[:, None]).astype(jnp.bfloat16)

    bf = jnp.bfloat16
    W1a = W_e1[:d].astype(bf)
    W1b = W_e1[d:2 * d].astype(bf)
    w1c = W_e1[2 * d:2 * d + 1]
    be1 = b_e1[None, :]
    We2 = W_e2.astype(bf)
    be2 = b_e2[None, :]
    Wn1a = W_n1[:d].astype(bf)
    Wn1b = W_n1[d:d + m_dim].astype(bf)
    bn1 = b_n1[None, :]
    Wn2 = W_n2.astype(bf)
    bn2 = b_n2[None, :]
    Wcx1 = jnp.concatenate([W_c1, W_x1], axis=1).astype(bf)
    bcx1 = jnp.concatenate([b_c1, b_x1])[None, :]
    zeros = jnp.zeros_like(W_c2)
    Wcx2 = jnp.concatenate(
        [jnp.concatenate([W_c2, zeros], axis=1),
         jnp.concatenate([zeros, W_x2], axis=1)], axis=0).astype(bf)
    bcx2 = jnp.concatenate([b_c2, b_x2])[None, :]

    full = lambda shp: pl.BlockSpec(shp, lambda bi, ii: (0,) * len(shp))

    grid = (b, n // R)
    node_out, coors_out = pl.pallas_call(
        _egnn_block,
        grid=grid,
        in_specs=[
            pl.BlockSpec((1, n, d), lambda bi, ii: (bi, 0, 0)),
            pl.BlockSpec((1, n, 3), lambda bi, ii: (bi, 0, 0)),
            pl.BlockSpec((1, 3, n), lambda bi, ii: (bi, 0, 0)),
            full(pool.shape),
            full(W1a.shape), full(W1b.shape), full(w1c.shape), full(be1.shape),
            full(We2.shape), full(be2.shape),
            full(Wn1a.shape), full(Wn1b.shape), full(bn1.shape),
            full(Wn2.shape), full(bn2.shape),
            full(Wcx1.shape), full(bcx1.shape), full(Wcx2.shape),
            full(bcx2.shape),
        ],
        out_specs=[
            pl.BlockSpec((1, R, d), lambda bi, ii: (bi, ii, 0)),
            pl.BlockSpec((1, R, 3), lambda bi, ii: (bi, ii, 0)),
        ],
        out_shape=[
            jax.ShapeDtypeStruct((b, n, d), jnp.float32),
            jax.ShapeDtypeStruct((b, n, 3), jnp.float32),
        ],
    )(feats, coors, coors_t, pool, W1a, W1b, w1c, be1, We2, be2,
      Wn1a, Wn1b, bn1, Wn2, bn2, Wcx1, bcx1, Wcx2, bcx2)
    return node_out, coors_out
